# baseline (speedup 1.0000x reference)
import functools

import jax
import jax.numpy as jnp
from jax import lax
from jax.experimental import pallas as pl
from jax.experimental.pallas import tpu as pltpu
from jax.experimental.pallas import tpu_sc as plsc

B, C, H, W, K = 32, 4, 128, 128, 128
HW = H * W
L = 16


def _rsqrt_plus_eps_inv(s):
    xi = plsc.bitcast(s, jnp.int32)
    yi = jnp.int32(0x5F3759DF) - lax.shift_right_logical(xi, 1)
    y = plsc.bitcast(yi, jnp.float32)
    for _ in range(3):
        y = y * (1.5 - 0.5 * s * y * y)
    sqrt_s = s * y
    return 1.0 / (sqrt_s + 1e-10)


def _sc_distances(feat, tp, tq):
    info = plsc.get_sparse_core_info()
    nc = info.num_cores
    mesh = plsc.VectorSubcoreMesh(core_axis_name="c", subcore_axis_name="s")

    @functools.partial(
        pl.kernel,
        mesh=mesh,
        out_type=(
            jax.ShapeDtypeStruct((B, K), jnp.float32),
            jax.ShapeDtypeStruct((B, K), jnp.float32),
        ),
        compiler_params=pltpu.CompilerParams(needs_layout_passes=False),
        scratch_types=[
            pltpu.VMEM((C * H, W), jnp.float32),
            pltpu.VMEM_SHARED((16, 2, C * H // 4, W), jnp.float32),
            pltpu.VMEM((2, K), jnp.int32),
            pltpu.VMEM((2, K), jnp.int32),
            pltpu.VMEM((K,), jnp.float32),
            pltpu.VMEM((K,), jnp.float32),
            pltpu.SemaphoreType.DMA,
            pltpu.SemaphoreType.DMA,
            pltpu.SemaphoreType.DMA,
            pltpu.SemaphoreType.DMA,
        ],
    )
    def body(feat_hbm, tp_hbm, tq_hbm, outp_hbm, outq_hbm,
             feat_v, sp_v, tp_v, tq_v, d2_v, pt_v, sa, sb, s2a, s2b):
        wid = lax.axis_index("s") * nc + lax.axis_index("c")
        sid = lax.axis_index("s")
        nsplit = 4
        rows = C * H // nsplit
        sems = (sa, sb)
        sems2 = (s2a, s2b)

        def h2s(m):
            return pltpu.async_copy(
                feat_hbm.at[pl.ds(wid * C * H + m * rows, rows)],
                sp_v.at[sid, m % 2],
                sems[m % 2],
            )

        def s2v(m):
            return pltpu.async_copy(
                sp_v.at[sid, m % 2],
                feat_v.at[pl.ds(m * rows, rows)],
                sems2[m % 2],
            )

        cps = {0: h2s(0), 1: h2s(1)}
        pltpu.sync_copy(tp_hbm.at[pl.ds(2 * wid, 2)], tp_v)
        pltpu.sync_copy(tq_hbm.at[pl.ds(2 * wid, 2)], tq_v)
        cps[0].wait()
        pulls = {0: s2v(0)}
        cps[1].wait()
        pulls[1] = s2v(1)
        pulls[0].wait()
        cps[2] = h2s(2)
        pulls[1].wait()
        cps[3] = h2s(3)
        cps[2].wait()
        pulls[2] = s2v(2)
        cps[3].wait()
        pulls[3] = s2v(3)
        pulls[2].wait()
        pulls[3].wait()

        def gather_norm(idx):
            row = lax.shift_right_logical(idx, 7)
            col = lax.bitwise_and(idx, W - 1)
            fs = [plsc.load_gather(feat_v, [row + c * H, col]) for c in range(C)]
            s = fs[0] * fs[0] + fs[1] * fs[1] + fs[2] * fs[2] + fs[3] * fs[3]
            r = _rsqrt_plus_eps_inv(s)
            return [f * r for f in fs]

        def chunk(j, _):
            sl = pl.ds(j * L, L)
            n0 = gather_norm(tp_v[0, sl])
            n1 = gather_norm(tp_v[1, sl])
            d2 = jnp.zeros((L,), jnp.float32)
            for a, b in zip(n0, n1):
                d = a - b
                d2 = d2 + d * d
            d2_v[sl] = d2

            p0 = gather_norm(tq_v[0, sl])
            p1 = gather_norm(tq_v[1, sl])
            l1 = jnp.zeros((L,), jnp.float32)
            for a, b in zip(p0, p1):
                l1 = l1 + jnp.abs(a - b)
            pt_v[sl] = jnp.maximum(1.0 - l1, 0.0)
            return 0

        lax.fori_loop(0, K // L, chunk, 0, unroll=False)

        pltpu.sync_copy(d2_v, outp_hbm.at[wid])
        pltpu.sync_copy(pt_v, outq_hbm.at[wid])

    return body(feat, tp, tq)


def _finalize_body(d2_ref, pt_ref, mp_ref, mq_ref, o_ref):
    mpf = mp_ref[...].astype(jnp.float32)
    mqf = mq_ref[...].astype(jnp.float32)
    ps = jnp.sum(d2_ref[...] * mpf)
    pc = jnp.sum(mpf)
    qs = jnp.sum(pt_ref[...] * mqf)
    qc = jnp.sum(mqf)
    loss = ps / (pc + 1e-4) + qs / (qc + 1e-4)
    o_ref[...] = jnp.full((1, 1), loss, jnp.float32)


def kernel(output, tag_pull, tag_push, mask_pull, mask_push):
    feat = output.reshape(B * C * H, W)
    tp = tag_pull.transpose(0, 2, 1).reshape(2 * B, K)
    tq = tag_push.transpose(0, 2, 1).reshape(2 * B, K)
    d2, pt = _sc_distances(feat, tp, tq)
    loss = pl.pallas_call(
        _finalize_body,
        out_shape=jax.ShapeDtypeStruct((1, 1), jnp.float32),
    )(d2, pt, mask_pull, mask_push)
    return loss[0, 0]


# speedup vs baseline: 1.1446x; 1.1446x over previous
import functools

import jax
import jax.numpy as jnp
from jax import lax
from jax.experimental import pallas as pl
from jax.experimental.pallas import tpu as pltpu
from jax.experimental.pallas import tpu_sc as plsc

B, C, H, W, K = 32, 4, 128, 128, 128
HW = H * W
L = 16


def _rsqrt_plus_eps_inv(s):
    xi = plsc.bitcast(s, jnp.int32)
    yi = jnp.int32(0x5F3759DF) - lax.shift_right_logical(xi, 1)
    y = plsc.bitcast(yi, jnp.float32)
    for _ in range(3):
        y = y * (1.5 - 0.5 * s * y * y)
    sqrt_s = s * y
    return 1.0 / (sqrt_s + 1e-10)


def _sc_distances(feat, tp, tq):
    info = plsc.get_sparse_core_info()
    nc = info.num_cores
    mesh = plsc.VectorSubcoreMesh(core_axis_name="c", subcore_axis_name="s")

    @functools.partial(
        pl.kernel,
        mesh=mesh,
        out_type=jax.ShapeDtypeStruct((B, 2 * K), jnp.float32),
        compiler_params=pltpu.CompilerParams(needs_layout_passes=False),
        scratch_types=[
            pltpu.VMEM((C * H, W), jnp.float32),
            pltpu.VMEM((2, K), jnp.int32),
            pltpu.VMEM((2, K), jnp.int32),
            pltpu.VMEM((2 * K,), jnp.float32),
            pltpu.SemaphoreType.DMA,
        ],
    )
    def body(feat_hbm, tp_hbm, tq_hbm, out_hbm,
             feat_v, tp_v, tq_v, res_v, sem):
        wid = lax.axis_index("s") * nc + lax.axis_index("c")
        cp = pltpu.async_copy(feat_hbm.at[pl.ds(wid * C * H, C * H)], feat_v, sem)
        pltpu.sync_copy(tp_hbm.at[pl.ds(2 * wid, 2)], tp_v)
        pltpu.sync_copy(tq_hbm.at[pl.ds(2 * wid, 2)], tq_v)
        cp.wait()

        def gather_norm(idx):
            row = lax.shift_right_logical(idx, 7)
            col = lax.bitwise_and(idx, W - 1)
            fs = [plsc.load_gather(feat_v, [row + c * H, col]) for c in range(C)]
            s = fs[0] * fs[0] + fs[1] * fs[1] + fs[2] * fs[2] + fs[3] * fs[3]
            r = _rsqrt_plus_eps_inv(s)
            return [f * r for f in fs]

        def chunk(j, _):
            sl = pl.ds(j * L, L)
            n0 = gather_norm(tp_v[0, sl])
            n1 = gather_norm(tp_v[1, sl])
            d2 = jnp.zeros((L,), jnp.float32)
            for a, b in zip(n0, n1):
                d = a - b
                d2 = d2 + d * d
            res_v[sl] = d2

            p0 = gather_norm(tq_v[0, sl])
            p1 = gather_norm(tq_v[1, sl])
            l1 = jnp.zeros((L,), jnp.float32)
            for a, b in zip(p0, p1):
                l1 = l1 + jnp.abs(a - b)
            res_v[pl.ds(K + j * L, L)] = jnp.maximum(1.0 - l1, 0.0)
            return 0

        lax.fori_loop(0, K // L, chunk, 0, unroll=False)

        pltpu.sync_copy(res_v, out_hbm.at[wid])

    return body(feat, tp, tq)


def _finalize_body(res_ref, mp_ref, mq_ref, o_ref):
    mpf = mp_ref[...].astype(jnp.float32)
    mqf = mq_ref[...].astype(jnp.float32)
    v = res_ref[...]
    ps = jnp.sum(v[:, :K] * mpf)
    pc = jnp.sum(mpf)
    qs = jnp.sum(v[:, K:] * mqf)
    qc = jnp.sum(mqf)
    loss = ps / (pc + 1e-4) + qs / (qc + 1e-4)
    o_ref[...] = jnp.full((1, 1), loss, jnp.float32)


def kernel(output, tag_pull, tag_push, mask_pull, mask_push):
    feat = output.reshape(B * C * H, W)
    tp = tag_pull.transpose(0, 2, 1).reshape(2 * B, K)
    tq = tag_push.transpose(0, 2, 1).reshape(2 * B, K)
    res = _sc_distances(feat, tp, tq)
    loss = pl.pallas_call(
        _finalize_body,
        out_shape=jax.ShapeDtypeStruct((1, 1), jnp.float32),
    )(res, mask_pull, mask_push)
    return loss[0, 0]
